# Initial kernel scaffold; baseline (speedup 1.0000x reference)
import jax
import jax.numpy as jnp
from jax.experimental import pallas as pl


def kernel(x, edge_index, W1, a_src1, a_dst1, b1, W2, a_src2, a_dst2, b2, Wf1, bf1, Wf2, bf2, Wf3, bf3):
    raise NotImplementedError("write your pallas kernel here")


# speedup vs baseline: 36.7723x; 36.7723x over previous
import jax
import jax.numpy as jnp
from jax import lax
from jax.experimental import pallas as pl
from jax.experimental.pallas import tpu as pltpu
from jax.experimental.pallas import tpu_sc as plsc

N = 10000
E = 320000
F_IN = 128
H1, C1 = 8, 16
H2, C2 = 1, 8

NW = 32
EPW = E // NW
K = 80
NB = EPW // K
NPAD = 10240
RPS = NPAD // 16

D1 = 144
D2 = 16


def _elu(x):
    return jnp.where(x > 0, x, jnp.exp(x) - 1.0)


def _full(v):
    return jnp.full((16,), v, dtype=jnp.int32)


def _tc1_body(x_ref, w1_ref, as_ref, ad_ref, hs_ref, at_ref):
    h = jnp.dot(x_ref[...], w1_ref[...], preferred_element_type=jnp.float32)
    asrc = jnp.dot(h, as_ref[...], preferred_element_type=jnp.float32)
    adst = jnp.dot(h, ad_ref[...], preferred_element_type=jnp.float32)
    pad = jnp.zeros((h.shape[0], 8), dtype=jnp.float32)
    hs_ref[...] = jnp.concatenate([h, asrc, pad], axis=1)
    at_ref[...] = jnp.concatenate([adst, pad], axis=1)


def _tc1(x, W1, As, Ad):
    blk = 2000
    return pl.pallas_call(
        _tc1_body,
        grid=(N // blk,),
        in_specs=[
            pl.BlockSpec((blk, F_IN), lambda i: (i, 0)),
            pl.BlockSpec((F_IN, F_IN), lambda i: (0, 0)),
            pl.BlockSpec((F_IN, H1), lambda i: (0, 0)),
            pl.BlockSpec((F_IN, H1), lambda i: (0, 0)),
        ],
        out_specs=[
            pl.BlockSpec((blk, D1), lambda i: (i, 0)),
            pl.BlockSpec((blk, D2), lambda i: (i, 0)),
        ],
        out_shape=[
            jax.ShapeDtypeStruct((N, D1), jnp.float32),
            jax.ShapeDtypeStruct((N, D2), jnp.float32),
        ],
    )(x, W1, As, Ad)


def _sc1_body(hs_hbm, at_hbm, ei_hbm, out_hbm,
              acc, buf0, buf1, dbuf0, dbuf1, ev0, ev1,
              gsA, gsB, dsA, dsB):
    c = lax.axis_index("c")
    s = lax.axis_index("s")
    wid = c * 16 + s
    base = wid * EPW
    row0 = s * RPS

    def _zr(r, carry):
        for cc in range(D1 // 16):
            buf0[r, pl.ds(cc * 16, 16)] = jnp.zeros((16,), jnp.float32)
        return carry
    lax.fori_loop(0, K, _zr, 0)
    for j in range(RPS // K):
        pltpu.sync_copy(buf0, acc.at[pl.ds(row0 + j * K, K)])
    plsc.subcore_barrier()

    iota = jnp.arange(16, dtype=jnp.int32)

    def fire(g, ev, buf, dbuf, gs, ds):
        off = base + g * K
        pltpu.sync_copy(ei_hbm.at[:, pl.ds(off, K)], ev)
        pltpu.async_copy(hs_hbm.at[ev.at[0]], buf, gs)
        pltpu.async_copy(at_hbm.at[ev.at[1]], dbuf, ds)

    def wait_g(ev, buf, dbuf, gs, ds):
        pltpu.make_async_copy(hs_hbm.at[ev.at[0]], buf, gs).wait()
        pltpu.make_async_copy(at_hbm.at[ev.at[1]], dbuf, ds).wait()

    def compute(buf, dbuf):
        for b in range(K // 16):
            rowv = iota + b * 16
            ws = []
            for h in range(H1):
                asrc = plsc.load_gather(buf, [rowv, _full(128 + h)])
                adst = plsc.load_gather(dbuf, [rowv, _full(h)])
                e = asrc + adst
                e = jnp.maximum(e, 0.2 * e)
                ws.append(jnp.exp(e))
            for h in range(H1):
                plsc.store_scatter(buf, [rowv, _full(128 + h)], ws[h])
            for h in range(H1):
                for cc in range(C1):
                    col = _full(16 * h + cc)
                    v = plsc.load_gather(buf, [rowv, col])
                    plsc.store_scatter(buf, [rowv, col], v * ws[h])

    fire(0, ev0, buf0, dbuf0, gsA, dsA)

    def pair(i, carry):
        g = 2 * i
        fire(g + 1, ev1, buf1, dbuf1, gsB, dsB)
        wait_g(ev0, buf0, dbuf0, gsA, dsA)
        compute(buf0, dbuf0)
        pltpu.sync_copy(buf0, acc.at[ev0.at[1]], add=True)
        fire(g + 2, ev0, buf0, dbuf0, gsA, dsA)
        wait_g(ev1, buf1, dbuf1, gsB, dsB)
        compute(buf1, dbuf1)
        pltpu.sync_copy(buf1, acc.at[ev1.at[1]], add=True)
        return carry

    lax.fori_loop(0, (NB - 1) // 2, pair, 0)
    wait_g(ev0, buf0, dbuf0, gsA, dsA)
    compute(buf0, dbuf0)
    pltpu.sync_copy(buf0, acc.at[ev0.at[1]], add=True)
    plsc.subcore_barrier()

    for j in range(RPS // 80):
        rs = row0 + j * 80

        @pl.when(rs < N)
        def _():
            pltpu.sync_copy(acc.at[pl.ds(rs, 80)],
                            out_hbm.at[c, pl.ds(rs, 80)])


def _sc1(hs1, adst1, edge_index):
    mesh = plsc.VectorSubcoreMesh(core_axis_name="c", subcore_axis_name="s")
    f = pl.kernel(
        _sc1_body,
        out_type=jax.ShapeDtypeStruct((2, N, D1), jnp.float32),
        mesh=mesh,
        compiler_params=pltpu.CompilerParams(
            use_tc_tiling_on_sc=False, needs_layout_passes=False),
        scratch_types=[
            pltpu.VMEM_SHARED((NPAD, D1), jnp.float32),
            pltpu.VMEM((K, D1), jnp.float32),
            pltpu.VMEM((K, D1), jnp.float32),
            pltpu.VMEM((K, D2), jnp.float32),
            pltpu.VMEM((K, D2), jnp.float32),
            pltpu.VMEM((2, K), jnp.int32),
            pltpu.VMEM((2, K), jnp.int32),
            pltpu.SemaphoreType.DMA,
            pltpu.SemaphoreType.DMA,
            pltpu.SemaphoreType.DMA,
            pltpu.SemaphoreType.DMA,
        ],
    )
    return f(hs1, adst1, edge_index)


def _tc2_body(p_ref, b1_ref, r_ref, w2e_ref, hs2_ref, a2_ref):
    acc = p_ref[0] + p_ref[1]
    num = acc[:, 0:128]
    den = jnp.dot(acc[:, 128:136], r_ref[...],
                  preferred_element_type=jnp.float32)
    ho = _elu(num / (den + 1e-16) + b1_ref[...])
    he = jnp.dot(ho, w2e_ref[...], preferred_element_type=jnp.float32)
    nrow = he.shape[0]
    ones = jnp.ones((nrow, 1), dtype=jnp.float32)
    pad = jnp.zeros((nrow, 6), dtype=jnp.float32)
    hs2_ref[...] = jnp.concatenate(
        [he[:, 0:8], ones, he[:, 8:9], pad], axis=1)
    a2_ref[...] = he[:, 9:10]


def _tc2(parts1, b1, R, W2e):
    blk = 2000
    return pl.pallas_call(
        _tc2_body,
        grid=(N // blk,),
        in_specs=[
            pl.BlockSpec((2, blk, D1), lambda i: (0, i, 0)),
            pl.BlockSpec((1, 128), lambda i: (0, 0)),
            pl.BlockSpec((H1, 128), lambda i: (0, 0)),
            pl.BlockSpec((128, 10), lambda i: (0, 0)),
        ],
        out_specs=[
            pl.BlockSpec((blk, D2), lambda i: (i, 0)),
            pl.BlockSpec((blk, 1), lambda i: (i, 0)),
        ],
        out_shape=[
            jax.ShapeDtypeStruct((N, D2), jnp.float32),
            jax.ShapeDtypeStruct((N, 1), jnp.float32),
        ],
    )(parts1, b1, R, W2e)


def _sc2_body(hs_hbm, at_hbm, ei_hbm, out_hbm,
              acc, atab, buf0, buf1, ev0, ev1, gsA, gsB):
    c = lax.axis_index("c")
    s = lax.axis_index("s")
    wid = c * 16 + s
    base = wid * EPW
    row0 = s * RPS

    pltpu.sync_copy(at_hbm, atab)

    def _zr(r, carry):
        buf0[r, pl.ds(0, 16)] = jnp.zeros((16,), jnp.float32)
        return carry
    lax.fori_loop(0, K, _zr, 0)
    for j in range(RPS // K):
        pltpu.sync_copy(buf0, acc.at[pl.ds(row0 + j * K, K)])
    plsc.subcore_barrier()

    iota = jnp.arange(16, dtype=jnp.int32)

    def fire(g, ev, buf, gs):
        off = base + g * K
        pltpu.sync_copy(ei_hbm.at[:, pl.ds(off, K)], ev)
        pltpu.async_copy(hs_hbm.at[ev.at[0]], buf, gs)

    def wait_g(ev, buf, gs):
        pltpu.make_async_copy(hs_hbm.at[ev.at[0]], buf, gs).wait()

    def compute(buf, ev):
        for b in range(K // 16):
            rowv = iota + b * 16
            dv = ev[1, pl.ds(b * 16, 16)]
            asrc = plsc.load_gather(buf, [rowv, _full(9)])
            adst = plsc.load_gather(atab, [dv])
            e = asrc + adst
            e = jnp.maximum(e, 0.2 * e)
            w = jnp.exp(e)
            for cc in range(9):
                col = _full(cc)
                v = plsc.load_gather(buf, [rowv, col])
                plsc.store_scatter(buf, [rowv, col], v * w)

    fire(0, ev0, buf0, gsA)

    def pair(i, carry):
        g = 2 * i
        fire(g + 1, ev1, buf1, gsB)
        wait_g(ev0, buf0, gsA)
        compute(buf0, ev0)
        pltpu.sync_copy(buf0, acc.at[ev0.at[1]], add=True)
        fire(g + 2, ev0, buf0, gsA)
        wait_g(ev1, buf1, gsB)
        compute(buf1, ev1)
        pltpu.sync_copy(buf1, acc.at[ev1.at[1]], add=True)
        return carry

    lax.fori_loop(0, (NB - 1) // 2, pair, 0)
    wait_g(ev0, buf0, gsA)
    compute(buf0, ev0)
    pltpu.sync_copy(buf0, acc.at[ev0.at[1]], add=True)
    plsc.subcore_barrier()

    for j in range(RPS // 80):
        rs = row0 + j * 80

        @pl.when(rs < N)
        def _():
            pltpu.sync_copy(acc.at[pl.ds(rs, 80)],
                            out_hbm.at[c, pl.ds(rs, 80)])


def _sc2(hs2, a2, edge_index):
    mesh = plsc.VectorSubcoreMesh(core_axis_name="c", subcore_axis_name="s")
    f = pl.kernel(
        _sc2_body,
        out_type=jax.ShapeDtypeStruct((2, N, D2), jnp.float32),
        mesh=mesh,
        compiler_params=pltpu.CompilerParams(
            use_tc_tiling_on_sc=False, needs_layout_passes=False),
        scratch_types=[
            pltpu.VMEM_SHARED((NPAD, D2), jnp.float32),
            pltpu.VMEM((N,), jnp.float32),
            pltpu.VMEM((K, D2), jnp.float32),
            pltpu.VMEM((K, D2), jnp.float32),
            pltpu.VMEM((2, K), jnp.int32),
            pltpu.VMEM((2, K), jnp.int32),
            pltpu.SemaphoreType.DMA,
            pltpu.SemaphoreType.DMA,
        ],
    )
    return f(hs2, a2, edge_index)


def _tc3a_body(p_ref, b2_ref, h_ref):
    acc = p_ref[0] + p_ref[1]
    num = acc[:, 0:8]
    den = acc[:, 8:9]
    h_ref[...] = _elu(num / (den + 1e-16) + b2_ref[...])


def _tc3a(parts2, b2):
    blk = 2000
    return pl.pallas_call(
        _tc3a_body,
        grid=(N // blk,),
        in_specs=[
            pl.BlockSpec((2, blk, D2), lambda i: (0, i, 0)),
            pl.BlockSpec((1, 8), lambda i: (0, 0)),
        ],
        out_specs=pl.BlockSpec((blk, 8), lambda i: (i, 0)),
        out_shape=jax.ShapeDtypeStruct((N, 8), jnp.float32),
    )(parts2, b2)


def _tc3b_body(z_ref, wf1_ref, bf1_ref, wf2_ref, bf2_ref, wf3_ref, bf3_ref,
               out_ref, accr):
    i = pl.program_id(0)

    @pl.when(i == 0)
    def _():
        accr[...] = jnp.zeros_like(accr)

    accr[...] += jnp.dot(z_ref[...], wf1_ref[...],
                         preferred_element_type=jnp.float32)

    @pl.when(i == pl.num_programs(0) - 1)
    def _():
        z1 = _elu(accr[...] + bf1_ref[...])
        z2 = _elu(jnp.dot(z1, wf2_ref[...],
                          preferred_element_type=jnp.float32) + bf2_ref[...])
        out_ref[...] = jnp.dot(z2, wf3_ref[...],
                               preferred_element_type=jnp.float32) + bf3_ref[...]


def _tc3b(zfull, Wf1, bf1, Wf2, bf2, Wf3, bf3):
    kb = 16000
    return pl.pallas_call(
        _tc3b_body,
        grid=(N * 8 // kb,),
        in_specs=[
            pl.BlockSpec((1, kb), lambda i: (0, i)),
            pl.BlockSpec((kb, 84), lambda i: (i, 0)),
            pl.BlockSpec((1, 84), lambda i: (0, 0)),
            pl.BlockSpec((84, 24), lambda i: (0, 0)),
            pl.BlockSpec((1, 24), lambda i: (0, 0)),
            pl.BlockSpec((24, 2), lambda i: (0, 0)),
            pl.BlockSpec((1, 2), lambda i: (0, 0)),
        ],
        out_specs=pl.BlockSpec((1, 2), lambda i: (0, 0)),
        out_shape=jax.ShapeDtypeStruct((1, 2), jnp.float32),
        scratch_shapes=[pltpu.VMEM((1, 84), jnp.float32)],
    )(zfull, Wf1, bf1, Wf2, bf2, Wf3, bf3)


def kernel(x, edge_index, W1, a_src1, a_dst1, b1, W2, a_src2, a_dst2, b2,
           Wf1, bf1, Wf2, bf2, Wf3, bf3):
    eye = jnp.eye(H1, dtype=jnp.float32)
    As = (eye[:, None, :] * a_src1[:, :, None]).reshape(F_IN, H1)
    Ad = (eye[:, None, :] * a_dst1[:, :, None]).reshape(F_IN, H1)
    R = jnp.repeat(eye, C1, axis=1)
    W2e = jnp.concatenate(
        [W2, W2 @ a_src2.reshape(8, 1), W2 @ a_dst2.reshape(8, 1)], axis=1)

    hs1, adst1 = _tc1(x, W1, As, Ad)
    parts1 = _sc1(hs1, adst1, edge_index)
    hs2, a2 = _tc2(parts1, b1.reshape(1, 128), R, W2e)
    parts2 = _sc2(hs2, a2.reshape(N), edge_index)
    h = _tc3a(parts2, b2.reshape(1, 8))
    logits = _tc3b(h.reshape(1, N * 8), Wf1, bf1.reshape(1, 84),
                   Wf2, bf2.reshape(1, 24), Wf3, bf3.reshape(1, 2))
    reg = jnp.zeros((1,), dtype=jnp.float32)
    return (logits, reg)


# speedup vs baseline: 45.7052x; 1.2429x over previous
import jax
import jax.numpy as jnp
from jax import lax
from jax.experimental import pallas as pl
from jax.experimental.pallas import tpu as pltpu
from jax.experimental.pallas import tpu_sc as plsc

N = 10000
E = 320000
F_IN = 128
H1, C1 = 8, 16
H2, C2 = 1, 8

NW = 32
EPW = E // NW
K = 80
NB = EPW // K
NPAD = 10240
RPS = NPAD // 16

D1 = 144
D2 = 16


def _elu(x):
    return jnp.where(x > 0, x, jnp.exp(x) - 1.0)


def _full(v):
    return jnp.full((16,), v, dtype=jnp.int32)


def _tc1_body(x_ref, w1_ref, as_ref, ad_ref, hs_ref, at_ref):
    h = jnp.dot(x_ref[...], w1_ref[...], preferred_element_type=jnp.float32)
    asrc = jnp.dot(h, as_ref[...], preferred_element_type=jnp.float32)
    adst = jnp.dot(h, ad_ref[...], preferred_element_type=jnp.float32)
    pad = jnp.zeros((h.shape[0], 8), dtype=jnp.float32)
    hs_ref[...] = jnp.concatenate([h, asrc, pad], axis=1)
    at_ref[...] = jnp.concatenate([adst, pad], axis=1)


def _tc1(x, W1, As, Ad):
    blk = 2000
    return pl.pallas_call(
        _tc1_body,
        grid=(N // blk,),
        in_specs=[
            pl.BlockSpec((blk, F_IN), lambda i: (i, 0)),
            pl.BlockSpec((F_IN, F_IN), lambda i: (0, 0)),
            pl.BlockSpec((F_IN, H1), lambda i: (0, 0)),
            pl.BlockSpec((F_IN, H1), lambda i: (0, 0)),
        ],
        out_specs=[
            pl.BlockSpec((blk, D1), lambda i: (i, 0)),
            pl.BlockSpec((blk, D2), lambda i: (i, 0)),
        ],
        out_shape=[
            jax.ShapeDtypeStruct((N, D1), jnp.float32),
            jax.ShapeDtypeStruct((N, D2), jnp.float32),
        ],
    )(x, W1, As, Ad)


def _sc1_body(hs_hbm, at_hbm, ei_hbm, out_hbm,
              acc, buf0, buf1, dbuf0, dbuf1, obuf, ev0, ev1,
              gsA, gsB, dsA, dsB):
    c = lax.axis_index("c")
    s = lax.axis_index("s")
    wid = c * 16 + s
    base = wid * EPW
    row0 = s * RPS

    def _zr(r, carry):
        for cc in range(D1 // 16):
            obuf[r, pl.ds(cc * 16, 16)] = jnp.zeros((16,), jnp.float32)
        return carry
    lax.fori_loop(0, K, _zr, 0)
    for j in range(RPS // K):
        pltpu.sync_copy(obuf, acc.at[pl.ds(row0 + j * K, K)])
    plsc.subcore_barrier()

    iota = jnp.arange(16, dtype=jnp.int32)

    def fire(g, ev, buf, dbuf, gs, ds):
        off = base + g * K
        pltpu.sync_copy(ei_hbm.at[:, pl.ds(off, K)], ev)
        pltpu.async_copy(hs_hbm.at[ev.at[0]], buf, gs)
        pltpu.async_copy(at_hbm.at[ev.at[1]], dbuf, ds)

    def wait_g(ev, buf, dbuf, gs, ds):
        pltpu.make_async_copy(hs_hbm.at[ev.at[0]], buf, gs).wait()
        pltpu.make_async_copy(at_hbm.at[ev.at[1]], dbuf, ds).wait()

    def compute(buf, dbuf):
        def sub(b, carry):
            rowv = iota + b * 16
            ws = []
            for h in range(H1):
                asrc = plsc.load_gather(buf, [rowv, _full(128 + h)])
                adst = plsc.load_gather(dbuf, [rowv, _full(h)])
                e = asrc + adst
                e = jnp.maximum(e, 0.2 * e)
                ws.append(jnp.exp(e))
            for h in range(H1):
                plsc.store_scatter(obuf, [rowv, _full(128 + h)], ws[h])
            for h in range(H1):
                for cc in range(C1):
                    col = _full(16 * h + cc)
                    v = plsc.load_gather(buf, [rowv, col])
                    plsc.store_scatter(obuf, [rowv, col], v * ws[h])
            return carry
        lax.fori_loop(0, K // 16, sub, 0)

    fire(0, ev0, buf0, dbuf0, gsA, dsA)

    def pair(i, carry):
        g = 2 * i
        fire(g + 1, ev1, buf1, dbuf1, gsB, dsB)
        wait_g(ev0, buf0, dbuf0, gsA, dsA)
        compute(buf0, dbuf0)
        pltpu.sync_copy(obuf, acc.at[ev0.at[1]], add=True)
        fire(g + 2, ev0, buf0, dbuf0, gsA, dsA)
        wait_g(ev1, buf1, dbuf1, gsB, dsB)
        compute(buf1, dbuf1)
        pltpu.sync_copy(obuf, acc.at[ev1.at[1]], add=True)
        return carry

    lax.fori_loop(0, (NB - 1) // 2, pair, 0)
    wait_g(ev0, buf0, dbuf0, gsA, dsA)
    compute(buf0, dbuf0)
    pltpu.sync_copy(obuf, acc.at[ev0.at[1]], add=True)
    plsc.subcore_barrier()

    for j in range(RPS // 80):
        rs = row0 + j * 80

        @pl.when(rs < N)
        def _():
            pltpu.sync_copy(acc.at[pl.ds(rs, 80)],
                            out_hbm.at[c, pl.ds(rs, 80)])


def _sc1(hs1, adst1, edge_index):
    mesh = plsc.VectorSubcoreMesh(core_axis_name="c", subcore_axis_name="s")
    f = pl.kernel(
        _sc1_body,
        out_type=jax.ShapeDtypeStruct((2, N, D1), jnp.float32),
        mesh=mesh,
        compiler_params=pltpu.CompilerParams(
            use_tc_tiling_on_sc=False, needs_layout_passes=False),
        scratch_types=[
            pltpu.VMEM_SHARED((NPAD, D1), jnp.float32),
            pltpu.VMEM((K, D1), jnp.float32),
            pltpu.VMEM((K, D1), jnp.float32),
            pltpu.VMEM((K, D2), jnp.float32),
            pltpu.VMEM((K, D2), jnp.float32),
            pltpu.VMEM((K, D1), jnp.float32),
            pltpu.VMEM((2, K), jnp.int32),
            pltpu.VMEM((2, K), jnp.int32),
            pltpu.SemaphoreType.DMA,
            pltpu.SemaphoreType.DMA,
            pltpu.SemaphoreType.DMA,
            pltpu.SemaphoreType.DMA,
        ],
    )
    return f(hs1, adst1, edge_index)


def _tc2_body(p_ref, b1_ref, r_ref, w2e_ref, hs2_ref, a2_ref):
    acc = p_ref[0] + p_ref[1]
    num = acc[:, 0:128]
    den = jnp.dot(acc[:, 128:136], r_ref[...],
                  preferred_element_type=jnp.float32)
    ho = _elu(num / (den + 1e-16) + b1_ref[...])
    he = jnp.dot(ho, w2e_ref[...], preferred_element_type=jnp.float32)
    nrow = he.shape[0]
    ones = jnp.ones((nrow, 1), dtype=jnp.float32)
    pad = jnp.zeros((nrow, 6), dtype=jnp.float32)
    hs2_ref[...] = jnp.concatenate(
        [he[:, 0:8], ones, he[:, 8:9], pad], axis=1)
    a2_ref[...] = he[:, 9:10]


def _tc2(parts1, b1, R, W2e):
    blk = 2000
    return pl.pallas_call(
        _tc2_body,
        grid=(N // blk,),
        in_specs=[
            pl.BlockSpec((2, blk, D1), lambda i: (0, i, 0)),
            pl.BlockSpec((1, 128), lambda i: (0, 0)),
            pl.BlockSpec((H1, 128), lambda i: (0, 0)),
            pl.BlockSpec((128, 10), lambda i: (0, 0)),
        ],
        out_specs=[
            pl.BlockSpec((blk, D2), lambda i: (i, 0)),
            pl.BlockSpec((blk, 1), lambda i: (i, 0)),
        ],
        out_shape=[
            jax.ShapeDtypeStruct((N, D2), jnp.float32),
            jax.ShapeDtypeStruct((N, 1), jnp.float32),
        ],
    )(parts1, b1, R, W2e)


def _sc2_body(hs_hbm, at_hbm, ei_hbm, out_hbm,
              acc, atab, buf0, buf1, obuf, ev0, ev1, gsA, gsB):
    c = lax.axis_index("c")
    s = lax.axis_index("s")
    wid = c * 16 + s
    base = wid * EPW
    row0 = s * RPS

    pltpu.sync_copy(at_hbm, atab)

    def _zr(r, carry):
        obuf[r, pl.ds(0, 16)] = jnp.zeros((16,), jnp.float32)
        return carry
    lax.fori_loop(0, K, _zr, 0)
    for j in range(RPS // K):
        pltpu.sync_copy(obuf, acc.at[pl.ds(row0 + j * K, K)])
    plsc.subcore_barrier()

    iota = jnp.arange(16, dtype=jnp.int32)

    def fire(g, ev, buf, gs):
        off = base + g * K
        pltpu.sync_copy(ei_hbm.at[:, pl.ds(off, K)], ev)
        pltpu.async_copy(hs_hbm.at[ev.at[0]], buf, gs)

    def wait_g(ev, buf, gs):
        pltpu.make_async_copy(hs_hbm.at[ev.at[0]], buf, gs).wait()

    def compute(buf, ev):
        for b in range(K // 16):
            rowv = iota + b * 16
            dv = ev[1, pl.ds(b * 16, 16)]
            asrc = plsc.load_gather(buf, [rowv, _full(9)])
            adst = plsc.load_gather(atab, [dv])
            e = asrc + adst
            e = jnp.maximum(e, 0.2 * e)
            w = jnp.exp(e)
            for cc in range(9):
                col = _full(cc)
                v = plsc.load_gather(buf, [rowv, col])
                plsc.store_scatter(obuf, [rowv, col], v * w)

    fire(0, ev0, buf0, gsA)

    def pair(i, carry):
        g = 2 * i
        fire(g + 1, ev1, buf1, gsB)
        wait_g(ev0, buf0, gsA)
        compute(buf0, ev0)
        pltpu.sync_copy(obuf, acc.at[ev0.at[1]], add=True)
        fire(g + 2, ev0, buf0, gsA)
        wait_g(ev1, buf1, gsB)
        compute(buf1, ev1)
        pltpu.sync_copy(obuf, acc.at[ev1.at[1]], add=True)
        return carry

    lax.fori_loop(0, (NB - 1) // 2, pair, 0)
    wait_g(ev0, buf0, gsA)
    compute(buf0, ev0)
    pltpu.sync_copy(obuf, acc.at[ev0.at[1]], add=True)
    plsc.subcore_barrier()

    for j in range(RPS // 80):
        rs = row0 + j * 80

        @pl.when(rs < N)
        def _():
            pltpu.sync_copy(acc.at[pl.ds(rs, 80)],
                            out_hbm.at[c, pl.ds(rs, 80)])


def _sc2(hs2, a2, edge_index):
    mesh = plsc.VectorSubcoreMesh(core_axis_name="c", subcore_axis_name="s")
    f = pl.kernel(
        _sc2_body,
        out_type=jax.ShapeDtypeStruct((2, N, D2), jnp.float32),
        mesh=mesh,
        compiler_params=pltpu.CompilerParams(
            use_tc_tiling_on_sc=False, needs_layout_passes=False),
        scratch_types=[
            pltpu.VMEM_SHARED((NPAD, D2), jnp.float32),
            pltpu.VMEM((N,), jnp.float32),
            pltpu.VMEM((K, D2), jnp.float32),
            pltpu.VMEM((K, D2), jnp.float32),
            pltpu.VMEM((K, D2), jnp.float32),
            pltpu.VMEM((2, K), jnp.int32),
            pltpu.VMEM((2, K), jnp.int32),
            pltpu.SemaphoreType.DMA,
            pltpu.SemaphoreType.DMA,
        ],
    )
    return f(hs2, a2, edge_index)


def _tc3a_body(p_ref, b2_ref, h_ref):
    acc = p_ref[0] + p_ref[1]
    num = acc[:, 0:8]
    den = acc[:, 8:9]
    h_ref[...] = _elu(num / (den + 1e-16) + b2_ref[...])


def _tc3a(parts2, b2):
    blk = 2000
    return pl.pallas_call(
        _tc3a_body,
        grid=(N // blk,),
        in_specs=[
            pl.BlockSpec((2, blk, D2), lambda i: (0, i, 0)),
            pl.BlockSpec((1, 8), lambda i: (0, 0)),
        ],
        out_specs=pl.BlockSpec((blk, 8), lambda i: (i, 0)),
        out_shape=jax.ShapeDtypeStruct((N, 8), jnp.float32),
    )(parts2, b2)


def _tc3b_body(z_ref, wf1_ref, bf1_ref, wf2_ref, bf2_ref, wf3_ref, bf3_ref,
               out_ref, accr):
    i = pl.program_id(0)

    @pl.when(i == 0)
    def _():
        accr[...] = jnp.zeros_like(accr)

    accr[...] += jnp.dot(z_ref[...], wf1_ref[...],
                         preferred_element_type=jnp.float32)

    @pl.when(i == pl.num_programs(0) - 1)
    def _():
        z1 = _elu(accr[...] + bf1_ref[...])
        z2 = _elu(jnp.dot(z1, wf2_ref[...],
                          preferred_element_type=jnp.float32) + bf2_ref[...])
        out_ref[...] = jnp.dot(z2, wf3_ref[...],
                               preferred_element_type=jnp.float32) + bf3_ref[...]


def _tc3b(zfull, Wf1, bf1, Wf2, bf2, Wf3, bf3):
    kb = 16000
    return pl.pallas_call(
        _tc3b_body,
        grid=(N * 8 // kb,),
        in_specs=[
            pl.BlockSpec((1, kb), lambda i: (0, i)),
            pl.BlockSpec((kb, 84), lambda i: (i, 0)),
            pl.BlockSpec((1, 84), lambda i: (0, 0)),
            pl.BlockSpec((84, 24), lambda i: (0, 0)),
            pl.BlockSpec((1, 24), lambda i: (0, 0)),
            pl.BlockSpec((24, 2), lambda i: (0, 0)),
            pl.BlockSpec((1, 2), lambda i: (0, 0)),
        ],
        out_specs=pl.BlockSpec((1, 2), lambda i: (0, 0)),
        out_shape=jax.ShapeDtypeStruct((1, 2), jnp.float32),
        scratch_shapes=[pltpu.VMEM((1, 84), jnp.float32)],
    )(zfull, Wf1, bf1, Wf2, bf2, Wf3, bf3)


def kernel(x, edge_index, W1, a_src1, a_dst1, b1, W2, a_src2, a_dst2, b2,
           Wf1, bf1, Wf2, bf2, Wf3, bf3):
    eye = jnp.eye(H1, dtype=jnp.float32)
    As = (eye[:, None, :] * a_src1[:, :, None]).reshape(F_IN, H1)
    Ad = (eye[:, None, :] * a_dst1[:, :, None]).reshape(F_IN, H1)
    R = jnp.repeat(eye, C1, axis=1)
    W2e = jnp.concatenate(
        [W2, W2 @ a_src2.reshape(8, 1), W2 @ a_dst2.reshape(8, 1)], axis=1)

    hs1, adst1 = _tc1(x, W1, As, Ad)
    parts1 = _sc1(hs1, adst1, edge_index)
    hs2, a2 = _tc2(parts1, b1.reshape(1, 128), R, W2e)
    parts2 = _sc2(hs2, a2.reshape(N), edge_index)
    h = _tc3a(parts2, b2.reshape(1, 8))
    logits = _tc3b(h.reshape(1, N * 8), Wf1, bf1.reshape(1, 84),
                   Wf2, bf2.reshape(1, 24), Wf3, bf3.reshape(1, 2))
    reg = jnp.zeros((1,), dtype=jnp.float32)
    return (logits, reg)
